# baseline (speedup 1.0000x reference)
import jax
import jax.numpy as jnp
from jax import lax
from jax.experimental import pallas as pl


def _mm_body(x_ref, w_ref, o_ref):
    o_ref[...] = lax.dot_general(
        x_ref[...], w_ref[...],
        (((1,), (1,)), ((), ())),
        preferred_element_type=jnp.float32,
    )


def kernel(ts, wMat):
    B, T, C = ts.shape
    I = wMat.shape[0]
    x = ts.reshape(B * T, C)
    Tt = 2048
    grid = ((B * T) // Tt,)
    out = pl.pallas_call(
        _mm_body,
        grid=grid,
        in_specs=[
            pl.BlockSpec((Tt, C), lambda i: (i, 0)),
            pl.BlockSpec((I, C), lambda i: (0, 0)),
        ],
        out_specs=pl.BlockSpec((Tt, I), lambda i: (i, 0)),
        out_shape=jax.ShapeDtypeStruct((B * T, I), jnp.float32),
    )(x, wMat)
    return out.reshape(B, T, I)


# speedup vs baseline: 1.0584x; 1.0584x over previous
import jax
import jax.numpy as jnp
from jax import lax
from jax.experimental import pallas as pl


def _mm_body(x_ref, w_ref, o_ref):
    o_ref[...] = lax.dot_general(
        x_ref[...].astype(jnp.bfloat16), w_ref[...].astype(jnp.bfloat16),
        (((1,), (1,)), ((), ())),
        preferred_element_type=jnp.float32,
    )


def kernel(ts, wMat):
    B, T, C = ts.shape
    I = wMat.shape[0]
    x = ts.reshape(B * T, C)
    Tt = 4096
    grid = ((B * T) // Tt,)
    out = pl.pallas_call(
        _mm_body,
        grid=grid,
        in_specs=[
            pl.BlockSpec((Tt, C), lambda i: (i, 0)),
            pl.BlockSpec((I, C), lambda i: (0, 0)),
        ],
        out_specs=pl.BlockSpec((Tt, I), lambda i: (i, 0)),
        out_shape=jax.ShapeDtypeStruct((B * T, I), jnp.float32),
    )(x, wMat)
    return out.reshape(B, T, I)
